# baseline (speedup 1.0000x reference)
import functools

import jax
import jax.numpy as jnp
import numpy as np
from jax import lax
from jax.experimental import pallas as pl
from jax.experimental.pallas import tpu as pltpu
from jax.experimental.pallas import tpu_sc as plsc

SEQ = 200
DIM = 64
PDIM = 128
BATCH = 4096
ROWS = BATCH * SEQ
NC, NS, LANES = 2, 16, 16
NW = NC * NS
SEQ_PER_W = BATCH // NW
CHUNK = SEQ
N_CHUNKS = SEQ_PER_W
IDX_GRP = 100
GRPS = CHUNK // IDX_GRP
NIB = 4


def _pos_encoding():
    half = DIM // 2
    positions = np.arange(SEQ).reshape(SEQ, 1)
    depths = np.arange(half).reshape(1, half) / half
    angle_rates = 1 / 10000 ** depths
    angle_rads = positions * angle_rates
    return np.concatenate([np.sin(angle_rads), np.cos(angle_rads)], axis=-1).astype(np.float32)


def _body(idx_hbm, table_hbm, pos_hbm, out_hbm,
          r0, r1, o0, o1, ib0, ib1, ib2, ib3, pos_v,
          si0, si1, so0, so1, sx0, sx1, sx2, sx3):
    rows = (r0, r1)
    obuf = (o0, o1)
    ib = (ib0, ib1, ib2, ib3)
    sin = (si0, si1)
    sout = (so0, so1)
    semi = (sx0, sx1, sx2, sx3)
    wid = lax.axis_index("s") * NC + lax.axis_index("c")
    bat0 = wid * SEQ_PER_W
    grp0 = wid * (N_CHUNKS * GRPS)

    pltpu.sync_copy(pos_hbm, pos_v)

    def fire_idx(c, k):
        pltpu.async_copy(idx_hbm.at[pl.ds(grp0 + c * GRPS, GRPS)], ib[k], semi[k])

    def wait_idx(k):
        pltpu.make_async_copy(idx_hbm.at[pl.ds(0, GRPS)], ib[k], semi[k]).wait()

    def fire_gather(c, b, gib):
        for g in range(GRPS):
            pltpu.async_copy(
                table_hbm.at[ib[gib].at[g]],
                rows[b].at[pl.ds(g * IDX_GRP, IDX_GRP)],
                sin[b],
            )

    def wait_in(b):
        pltpu.make_async_copy(table_hbm.at[pl.ds(0, CHUNK)], rows[b], sin[b]).wait()

    def fire_out(c, b):
        pltpu.async_copy(obuf[b], out_hbm.at[bat0 + c], sout[b])

    def wait_out(b):
        pltpu.make_async_copy(obuf[b], out_hbm.at[0], sout[b]).wait()

    def compute(b, o):
        def row_body(r, carry):
            for j in range(2):
                for i in range(DIM // LANES):
                    obuf[o][2 * r + j, pl.ds(i * LANES, LANES)] = (
                        rows[b][2 * r + j, pl.ds(i * LANES, LANES)]
                        + pos_v[r, pl.ds(j * DIM + i * LANES, LANES)]
                    )
            return carry

        lax.fori_loop(0, SEQ // 2, row_body, 0)

    for k in range(NIB):
        pltpu.sync_copy(idx_hbm.at[pl.ds(grp0 + k * GRPS, GRPS)], ib[k])
    fire_gather(0, 0, 0)
    fire_gather(1, 1, 1)

    def step(c, b, kib, gib, head=False, idx_ahead=True, gather_ahead=True):
        wait_in(b)
        if idx_ahead:
            fire_idx(c + NIB, kib)
        if not head:
            wait_out(b)
        compute(b, b)
        fire_out(c, b)
        if gather_ahead:
            if not head:
                wait_idx(gib)
            fire_gather(c + 2, b, gib)

    step(0, 0, 0, 2, head=True)
    step(1, 1, 1, 3, head=True)

    def outer(t, carry):
        for j in range(NIB):
            step(2 + t * NIB + j, j % 2, (2 + j) % NIB, j)
        return carry

    lax.fori_loop(0, (N_CHUNKS - 8) // NIB, outer, 0)

    step(122, 0, 2, 0)
    step(123, 1, 3, 1)
    step(124, 0, 0, 2, idx_ahead=False)
    step(125, 1, 1, 3, idx_ahead=False)
    step(126, 0, 2, 0, idx_ahead=False, gather_ahead=False)
    step(127, 1, 3, 1, idx_ahead=False, gather_ahead=False)

    wait_out(0)
    wait_out(1)


@functools.partial(jax.jit, static_argnums=())
def _run(idx, table_pad, pos_packed):
    kern = pl.kernel(
        _body,
        out_type=jax.ShapeDtypeStruct((BATCH, SEQ, DIM), jnp.float32),
        mesh=plsc.VectorSubcoreMesh(core_axis_name="c", subcore_axis_name="s"),
        scratch_types=[
            pltpu.VMEM((CHUNK, PDIM), jnp.float32),
            pltpu.VMEM((CHUNK, PDIM), jnp.float32),
            pltpu.VMEM((CHUNK, DIM), jnp.float32),
            pltpu.VMEM((CHUNK, DIM), jnp.float32),
            pltpu.VMEM((GRPS, IDX_GRP), jnp.int32),
            pltpu.VMEM((GRPS, IDX_GRP), jnp.int32),
            pltpu.VMEM((GRPS, IDX_GRP), jnp.int32),
            pltpu.VMEM((GRPS, IDX_GRP), jnp.int32),
            pltpu.VMEM((SEQ // 2, PDIM), jnp.float32),
            pltpu.SemaphoreType.DMA,
            pltpu.SemaphoreType.DMA,
            pltpu.SemaphoreType.DMA,
            pltpu.SemaphoreType.DMA,
            pltpu.SemaphoreType.DMA,
            pltpu.SemaphoreType.DMA,
            pltpu.SemaphoreType.DMA,
            pltpu.SemaphoreType.DMA,
        ],
    )
    return kern(idx, table_pad, pos_packed)


def kernel(inputs, token_table):
    idx = inputs.astype(jnp.int32).reshape(ROWS // IDX_GRP, IDX_GRP)
    table_pad = jnp.pad(token_table, ((0, 0), (0, PDIM - DIM)))
    pos_packed = jnp.asarray(_pos_encoding().reshape(SEQ // 2, PDIM))
    return _run(idx, table_pad, pos_packed)


# speedup vs baseline: 1.1278x; 1.1278x over previous
import functools

import jax
import jax.numpy as jnp
import numpy as np
from jax import lax
from jax.experimental import pallas as pl
from jax.experimental.pallas import tpu as pltpu
from jax.experimental.pallas import tpu_sc as plsc

SEQ = 200
DIM = 64
PDIM = 128
BATCH = 4096
ROWS = BATCH * SEQ
NC, NS, LANES = 2, 16, 16
NW = NC * NS
SEQ_PER_W = BATCH // NW
CHUNK = SEQ
PROWS = CHUNK // 2
N_CHUNKS = SEQ_PER_W
IDX_GRP = 100
GRPS = CHUNK // IDX_GRP
GRP_PER_W = N_CHUNKS * GRPS
NRB = 3


def _pos_encoding():
    half = DIM // 2
    positions = np.arange(SEQ).reshape(SEQ, 1)
    depths = np.arange(half).reshape(1, half) / half
    angle_rates = 1 / 10000 ** depths
    angle_rads = positions * angle_rates
    return np.concatenate([np.sin(angle_rads), np.cos(angle_rads)], axis=-1).astype(np.float32)


def _body(idx_hbm, table_hbm, pos_hbm, out_hbm,
          idx_all, r0, r1, r2, o0, o1, pos_v,
          si0, si1, si2, so0, so1):
    rows = (r0, r1, r2)
    obuf = (o0, o1)
    sin = (si0, si1, si2)
    sout = (so0, so1)
    wid = lax.axis_index("s") * NC + lax.axis_index("c")
    prow0 = wid * (N_CHUNKS * PROWS)

    pltpu.sync_copy(pos_hbm, pos_v)
    pltpu.sync_copy(idx_hbm.at[pl.ds(wid * GRP_PER_W, GRP_PER_W)], idx_all)

    def fire_gather(c, b):
        for g in range(GRPS):
            pltpu.async_copy(
                table_hbm.at[idx_all.at[c * GRPS + g]],
                rows[b].at[pl.ds(g * IDX_GRP, IDX_GRP)],
                sin[b],
            )

    def wait_in(b):
        pltpu.make_async_copy(table_hbm.at[pl.ds(0, CHUNK)], rows[b], sin[b]).wait()

    def fire_out(c, o):
        pltpu.async_copy(obuf[o], out_hbm.at[pl.ds(prow0 + c * PROWS, PROWS)], sout[o])

    def wait_out(o):
        pltpu.make_async_copy(obuf[o], out_hbm.at[pl.ds(0, PROWS)], sout[o]).wait()

    def compute(b, o):
        def row_body(r, carry):
            for j in range(2):
                for i in range(DIM // LANES):
                    obuf[o][r, pl.ds(j * DIM + i * LANES, LANES)] = (
                        rows[b][2 * r + j, pl.ds(i * LANES, LANES)]
                        + pos_v[r, pl.ds(j * DIM + i * LANES, LANES)]
                    )
            return carry

        lax.fori_loop(0, PROWS, row_body, 0)

    def step(c, b, o, gb, head=False, gather_ahead=True):
        wait_in(b)
        if not head:
            wait_out(o)
        compute(b, o)
        fire_out(c, o)
        if gather_ahead:
            fire_gather(c + 2, gb)

    fire_gather(0, 0)
    fire_gather(1, 1)

    step(0, 0, 0, 2, head=True)
    step(1, 1, 1, 0, head=True)

    def outer(t, carry):
        for j in range(6):
            c = 2 + t * 6 + j
            step(c, (2 + j) % NRB, j % 2, (1 + j) % NRB)
        return carry

    lax.fori_loop(0, (N_CHUNKS - 8) // 6, outer, 0)

    for c in range(N_CHUNKS - 6, N_CHUNKS):
        step(c, c % NRB, c % 2, (c + 2) % NRB, gather_ahead=(c + 2 < N_CHUNKS))

    wait_out(0)
    wait_out(1)


@functools.partial(jax.jit, static_argnums=())
def _run(idx, table, pos_packed):
    kern = pl.kernel(
        _body,
        out_type=jax.ShapeDtypeStruct((ROWS // 2, PDIM), jnp.float32),
        mesh=plsc.VectorSubcoreMesh(core_axis_name="c", subcore_axis_name="s"),
        scratch_types=[
            pltpu.VMEM((GRP_PER_W, IDX_GRP), jnp.int32),
            pltpu.VMEM((CHUNK, DIM), jnp.float32),
            pltpu.VMEM((CHUNK, DIM), jnp.float32),
            pltpu.VMEM((CHUNK, DIM), jnp.float32),
            pltpu.VMEM((PROWS, PDIM), jnp.float32),
            pltpu.VMEM((PROWS, PDIM), jnp.float32),
            pltpu.VMEM((SEQ // 2, PDIM), jnp.float32),
            pltpu.SemaphoreType.DMA,
            pltpu.SemaphoreType.DMA,
            pltpu.SemaphoreType.DMA,
            pltpu.SemaphoreType.DMA,
            pltpu.SemaphoreType.DMA,
        ],
        compiler_params=pltpu.CompilerParams(use_tc_tiling_on_sc=False),
    )
    return kern(idx, table, pos_packed)


def kernel(inputs, token_table):
    idx = inputs.astype(jnp.int32).reshape(ROWS // IDX_GRP, IDX_GRP)
    pos_packed = jnp.asarray(_pos_encoding().reshape(SEQ // 2, PDIM))
    packed = _run(idx, token_table, pos_packed)
    return packed.reshape(BATCH, SEQ, DIM)
